# Initial kernel scaffold; baseline (speedup 1.0000x reference)
import functools

import jax
import jax.numpy as jnp
from jax import lax
from jax.experimental import pallas as pl
from jax.experimental.pallas import tpu as pltpu
from jax.experimental.pallas import tpu_sc as plsc

N = 10000
D = 128
NPAD = 10016
DUMMY = 10008
ROWS_PT = NPAD // 16

E = 320000
CH = 128
NCH = 79
EPT = NCH * CH
EPAD = 32 * EPT

ED = 200000
NCHD = 50
EDPT = NCHD * CH
EDPAD = 32 * EDPT

_mesh = plsc.VectorSubcoreMesh(core_axis_name="c", subcore_axis_name="s")


def _agg_body(with_cnt, *refs):
    if with_cnt:
        (src_hbm, dst_hbm, x_hbm, zeros_hbm, ones_hbm,
         agg_out, cnt_out,
         sidx, didx, rows, ones_v, agg_sh, cnt_sh, gsem) = refs
    else:
        (src_hbm, dst_hbm, x_hbm, zeros_hbm,
         agg_out,
         sidx, didx, rows, agg_sh, gsem) = refs
    c = lax.axis_index("c")
    s = lax.axis_index("s")
    rbase = s * ROWS_PT
    pltpu.sync_copy(zeros_hbm.at[pl.ds(rbase, ROWS_PT)],
                    agg_sh.at[pl.ds(rbase, ROWS_PT)])
    if with_cnt:
        pltpu.sync_copy(zeros_hbm.at[pl.ds(rbase, ROWS_PT), pl.ds(0, 16)],
                        cnt_sh.at[pl.ds(rbase, ROWS_PT)])
        pltpu.sync_copy(ones_hbm, ones_v)
    plsc.subcore_barrier()

    ebase = (c * 16 + s) * EPT

    def body(i, carry):
        off = pl.multiple_of(ebase + i * CH, 8)
        pltpu.sync_copy(src_hbm.at[pl.ds(off, CH)], sidx)
        pltpu.sync_copy(dst_hbm.at[pl.ds(off, CH)], didx)
        pltpu.async_copy(x_hbm.at[sidx], rows, gsem).wait()
        pltpu.sync_copy(rows, agg_sh.at[didx], add=True)
        if with_cnt:
            pltpu.sync_copy(ones_v, cnt_sh.at[didx], add=True)
        return carry

    lax.fori_loop(0, NCH, body, 0)
    plsc.subcore_barrier()
    pltpu.sync_copy(agg_sh.at[pl.ds(rbase, ROWS_PT)],
                    agg_out.at[c, pl.ds(rbase, ROWS_PT)])
    if with_cnt:
        pltpu.sync_copy(cnt_sh.at[pl.ds(rbase, ROWS_PT)],
                        cnt_out.at[c, pl.ds(rbase, ROWS_PT)])


_agg_cnt_call = pl.kernel(
    functools.partial(_agg_body, True),
    out_type=(jax.ShapeDtypeStruct((2, NPAD, D), jnp.float32),
              jax.ShapeDtypeStruct((2, NPAD, 16), jnp.float32)),
    mesh=_mesh,
    scratch_types=[
        pltpu.VMEM((CH,), jnp.int32),
        pltpu.VMEM((CH,), jnp.int32),
        pltpu.VMEM((CH, D), jnp.float32),
        pltpu.VMEM((CH, 16), jnp.float32),
        pltpu.VMEM_SHARED((NPAD, D), jnp.float32),
        pltpu.VMEM_SHARED((NPAD, 16), jnp.float32),
        pltpu.SemaphoreType.DMA,
    ],
)

_agg_call = pl.kernel(
    functools.partial(_agg_body, False),
    out_type=jax.ShapeDtypeStruct((2, NPAD, D), jnp.float32),
    mesh=_mesh,
    scratch_types=[
        pltpu.VMEM((CH,), jnp.int32),
        pltpu.VMEM((CH,), jnp.int32),
        pltpu.VMEM((CH, D), jnp.float32),
        pltpu.VMEM_SHARED((NPAD, D), jnp.float32),
        pltpu.SemaphoreType.DMA,
    ],
)


def _decode_body(es_hbm, ed_hbm, p_hbm, q_hbm, w2_hbm, b2_hbm,
                 out_hbm,
                 sidx, didx, prow, qrow, w2v, b2v, sc_v, sem1, sem2):
    c = lax.axis_index("c")
    s = lax.axis_index("s")
    ebase = (c * 16 + s) * EDPT
    pltpu.sync_copy(w2_hbm, w2v)
    pltpu.sync_copy(b2_hbm, b2v)
    w2r = [w2v[pl.ds(16 * k, 16)] for k in range(8)]
    b2r = b2v[...]
    lanes = lax.broadcasted_iota(jnp.int32, (16,), 0)

    def chunk(i, carry):
        off = pl.multiple_of(ebase + i * CH, 8)
        pltpu.sync_copy(es_hbm.at[pl.ds(off, CH)], sidx)
        pltpu.sync_copy(ed_hbm.at[pl.ds(off, CH)], didx)
        pltpu.async_copy(p_hbm.at[sidx], prow, sem1).wait()
        pltpu.async_copy(q_hbm.at[didx], qrow, sem2).wait()

        def group(g, carry2):
            vec = jnp.zeros((16,), jnp.float32)
            for j in range(16):
                e = g * 16 + j
                acc = None
                for k in range(8):
                    p = prow[e, pl.ds(16 * k, 16)]
                    q = qrow[e, pl.ds(16 * k, 16)]
                    t = jnp.maximum(p + q, 0.0) * w2r[k]
                    acc = t if acc is None else acc + t
                sval = jnp.sum(acc)
                vec = jnp.where(lanes == j, sval, vec)
            vec = vec + b2r
            vec = 1.0 / (1.0 + jnp.exp(-vec))
            sc_v[pl.ds(g * 16, 16)] = vec
            return carry2

        lax.fori_loop(0, CH // 16, group, 0)
        pltpu.sync_copy(sc_v, out_hbm.at[pl.ds(off, CH)])
        return carry

    lax.fori_loop(0, NCHD, chunk, 0)


_decode_call = pl.kernel(
    _decode_body,
    out_type=jax.ShapeDtypeStruct((EDPAD,), jnp.float32),
    mesh=_mesh,
    scratch_types=[
        pltpu.VMEM((CH,), jnp.int32),
        pltpu.VMEM((CH,), jnp.int32),
        pltpu.VMEM((CH, D), jnp.float32),
        pltpu.VMEM((CH, D), jnp.float32),
        pltpu.VMEM((D,), jnp.float32),
        pltpu.VMEM((16,), jnp.float32),
        pltpu.VMEM((CH,), jnp.float32),
        pltpu.SemaphoreType.DMA,
        pltpu.SemaphoreType.DMA,
    ],
)


def _mean_from_parts(agg_ref, cnt_ref):
    a = agg_ref[0, :N, :] + agg_ref[1, :N, :]
    cnt = cnt_ref[0, :N, 0:1] + cnt_ref[1, :N, 0:1]
    return a / jnp.maximum(cnt, 1.0)


def _tc_layer_body(agg_ref, cnt_ref, h_ref, wl_ref, bl_ref, wr_ref,
                   g_ref, be_ref, out_ref):
    mean = _mean_from_parts(agg_ref, cnt_ref)
    h = h_ref[...]
    y = (jnp.dot(mean, wl_ref[...], preferred_element_type=jnp.float32)
         + bl_ref[...]
         + jnp.dot(h, wr_ref[...], preferred_element_type=jnp.float32))
    mu = jnp.mean(y, axis=0, keepdims=True)
    dlt = y - mu
    var = jnp.mean(dlt * dlt, axis=0, keepdims=True)
    yn = dlt * lax.rsqrt(var + 1e-5) * g_ref[...] + be_ref[...]
    out_ref[...] = jnp.maximum(yn, 0.0) + h


_tc_layer_call = pl.pallas_call(
    _tc_layer_body,
    out_shape=jax.ShapeDtypeStruct((N, D), jnp.float32),
)


def _tc_final_body(agg_ref, cnt_ref, h_ref, wl_ref, bl_ref, wr_ref,
                   w1a_ref, w1b_ref, b1_ref,
                   z_ref, p_ref, q_ref):
    mean = _mean_from_parts(agg_ref, cnt_ref)
    z = (jnp.dot(mean, wl_ref[...], preferred_element_type=jnp.float32)
         + bl_ref[...]
         + jnp.dot(h_ref[...], wr_ref[...], preferred_element_type=jnp.float32))
    z_ref[...] = z
    p_ref[...] = jnp.dot(z, w1a_ref[...], preferred_element_type=jnp.float32)
    q_ref[...] = (jnp.dot(z, w1b_ref[...], preferred_element_type=jnp.float32)
                  + b1_ref[...])


_tc_final_call = pl.pallas_call(
    _tc_final_body,
    out_shape=(jax.ShapeDtypeStruct((N, D), jnp.float32),
               jax.ShapeDtypeStruct((N, D), jnp.float32),
               jax.ShapeDtypeStruct((N, D), jnp.float32)),
)


def kernel(x, edge_index, pos_edge_index, neg_edge_index,
           Wl0, bl0, Wr0, Wl1, bl1, Wr1, Wl2, bl2, Wr2,
           g0, be0, g1, be1, ep_W1, ep_b1, ep_W2, ep_b2):
    i32 = jnp.int32
    src = edge_index[0].astype(i32)
    dst = edge_index[1].astype(i32)
    src_p = jnp.concatenate([src, jnp.zeros((EPAD - E,), i32)])
    dst_p = jnp.concatenate([dst, jnp.full((EPAD - E,), DUMMY, i32)])
    zeros = jnp.zeros((NPAD, D), jnp.float32)
    ones = jnp.ones((CH, 16), jnp.float32)

    bl0r, bl1r, bl2r = bl0.reshape(1, D), bl1.reshape(1, D), bl2.reshape(1, D)
    g0r, be0r = g0.reshape(1, D), be0.reshape(1, D)
    g1r, be1r = g1.reshape(1, D), be1.reshape(1, D)

    agg0, cntp = _agg_cnt_call(src_p, dst_p, x, zeros, ones)
    h0 = _tc_layer_call(agg0, cntp, x, Wl0, bl0r, Wr0, g0r, be0r)
    agg1 = _agg_call(src_p, dst_p, h0, zeros)
    h1 = _tc_layer_call(agg1, cntp, h0, Wl1, bl1r, Wr1, g1r, be1r)
    agg2 = _agg_call(src_p, dst_p, h1, zeros)
    z, p, q = _tc_final_call(agg2, cntp, h1, Wl2, bl2r, Wr2,
                             ep_W1[:D, :], ep_W1[D:, :], ep_b1.reshape(1, D))

    es = jnp.concatenate([pos_edge_index[0], neg_edge_index[0],
                          jnp.zeros((EDPAD - ED,), i32)])
    ed = jnp.concatenate([pos_edge_index[1], neg_edge_index[1],
                          jnp.zeros((EDPAD - ED,), i32)])
    w2 = ep_W2[:, 0]
    b2v = jnp.broadcast_to(ep_b2, (16,)).astype(jnp.float32)
    scores = _decode_call(es, ed, p, q, w2, b2v)
    return z, scores[:100000], scores[100000:200000]


# speedup vs baseline: 2.4127x; 2.4127x over previous
import functools

import jax
import jax.numpy as jnp
from jax import lax
from jax.experimental import pallas as pl
from jax.experimental.pallas import tpu as pltpu
from jax.experimental.pallas import tpu_sc as plsc

N = 10000
D = 128
NPAD = 10112
DUMMY = 10016
ROWS_PT = NPAD // 16

E = 320000
CH = 128
NCH = 79
EPT = NCH * CH
EPAD = 32 * EPT

ED = 200000
NCHD = 50
EDPT = NCHD * CH
EDPAD = 32 * EDPT

_mesh = plsc.VectorSubcoreMesh(core_axis_name="c", subcore_axis_name="s")

_GDN = lax.GatherDimensionNumbers(
    offset_dims=(), collapsed_slice_dims=(0,), start_index_map=(0,))


def _permute(v, idx):
    return lax.gather(v, idx[:, None], _GDN, (1,),
                      mode=lax.GatherScatterMode.PROMISE_IN_BOUNDS)


def _agg_body(src_hbm, dst_hbm, x_hbm, zeros_hbm,
              agg_out,
              sidx, didx, rows, agg_sh, gsem):
    c = lax.axis_index("c")
    s = lax.axis_index("s")
    rbase = s * ROWS_PT
    chunks = ((0, 128), (128, 128), (256, 128), (384, 128), (512, 120))
    for off, n in chunks:
        pltpu.sync_copy(zeros_hbm.at[pl.ds(rbase + off, n)],
                        rows.at[pl.ds(0, n)])
        pltpu.sync_copy(rows.at[pl.ds(0, n)],
                        agg_sh.at[pl.ds(rbase + off, n)])
    plsc.subcore_barrier()

    ebase = (c * 16 + s) * EPT

    def body(i, carry):
        off = pl.multiple_of(ebase + i * CH, 8)
        pltpu.sync_copy(src_hbm.at[pl.ds(off, CH)], sidx)
        pltpu.sync_copy(dst_hbm.at[pl.ds(off, CH)], didx)
        pltpu.async_copy(x_hbm.at[sidx], rows, gsem).wait()
        pltpu.sync_copy(rows, agg_sh.at[didx], add=True)
        return carry

    lax.fori_loop(0, NCH, body, 0)
    plsc.subcore_barrier()
    for off, n in chunks:
        pltpu.sync_copy(agg_sh.at[pl.ds(rbase + off, n)],
                        rows.at[pl.ds(0, n)])
        pltpu.sync_copy(rows.at[pl.ds(0, n)],
                        agg_out.at[c, pl.ds(rbase + off, n)])


_agg_call = pl.kernel(
    _agg_body,
    out_type=jax.ShapeDtypeStruct((2, NPAD, D), jnp.float32),
    mesh=_mesh,
    scratch_types=[
        pltpu.VMEM((CH,), jnp.int32),
        pltpu.VMEM((CH,), jnp.int32),
        pltpu.VMEM((CH, D), jnp.float32),
        pltpu.VMEM_SHARED((NPAD, D), jnp.float32),
        pltpu.SemaphoreType.DMA,
    ],
)


def _decode_body(es_hbm, ed_hbm, p_hbm, q_hbm, w2_hbm, b2_hbm,
                 out_hbm,
                 sidx, didx, prow, qrow, w2v, b2v, sc_v, sem1, sem2):
    c = lax.axis_index("c")
    s = lax.axis_index("s")
    ebase = (c * 16 + s) * EDPT
    pltpu.sync_copy(w2_hbm, w2v)
    pltpu.sync_copy(b2_hbm, b2v)
    w2r = [w2v[pl.ds(16 * k, 16)] for k in range(8)]
    b2r = b2v[...]
    lanes = lax.broadcasted_iota(jnp.int32, (16,), 0)

    def chunk(i, carry):
        off = pl.multiple_of(ebase + i * CH, 8)
        pltpu.sync_copy(es_hbm.at[pl.ds(off, CH)], sidx)
        pltpu.sync_copy(ed_hbm.at[pl.ds(off, CH)], didx)
        pltpu.async_copy(p_hbm.at[sidx], prow, sem1).wait()
        pltpu.async_copy(q_hbm.at[didx], qrow, sem2).wait()

        def group(g, carry2):
            vec = jnp.zeros((16,), jnp.float32)
            for j in range(16):
                e = g * 16 + j
                acc = None
                for k in range(8):
                    p = prow[e, pl.ds(16 * k, 16)]
                    q = qrow[e, pl.ds(16 * k, 16)]
                    t = jnp.maximum(p + q, 0.0) * w2r[k]
                    acc = t if acc is None else acc + t
                for sh in (8, 4, 2, 1):
                    acc = acc + _permute(acc, lanes ^ sh)
                vec = jnp.where(lanes == j, acc, vec)
            vec = vec + b2r
            vec = 1.0 / (1.0 + jnp.exp(-vec))
            sc_v[pl.ds(g * 16, 16)] = vec
            return carry2

        lax.fori_loop(0, CH // 16, group, 0)
        pltpu.sync_copy(sc_v, out_hbm.at[pl.ds(off, CH)])
        return carry

    lax.fori_loop(0, NCHD, chunk, 0)


_decode_call = pl.kernel(
    _decode_body,
    out_type=jax.ShapeDtypeStruct((EDPAD,), jnp.float32),
    mesh=_mesh,
    scratch_types=[
        pltpu.VMEM((CH,), jnp.int32),
        pltpu.VMEM((CH,), jnp.int32),
        pltpu.VMEM((CH, D), jnp.float32),
        pltpu.VMEM((CH, D), jnp.float32),
        pltpu.VMEM((D,), jnp.float32),
        pltpu.VMEM((16,), jnp.float32),
        pltpu.VMEM((CH,), jnp.float32),
        pltpu.SemaphoreType.DMA,
        pltpu.SemaphoreType.DMA,
    ],
)


def _mean_from_parts(agg_ref, cnt_ref):
    a = agg_ref[0, :N, :] + agg_ref[1, :N, :]
    cnt = cnt_ref[0, :N, 0:1] + cnt_ref[1, :N, 0:1]
    return a / jnp.maximum(cnt, 1.0)


def _tc_layer_body(agg_ref, cnt_ref, h_ref, wl_ref, bl_ref, wr_ref,
                   g_ref, be_ref, out_ref):
    mean = _mean_from_parts(agg_ref, cnt_ref)
    h = h_ref[...]
    y = (jnp.dot(mean, wl_ref[...], preferred_element_type=jnp.float32)
         + bl_ref[...]
         + jnp.dot(h, wr_ref[...], preferred_element_type=jnp.float32))
    mu = jnp.mean(y, axis=0, keepdims=True)
    dlt = y - mu
    var = jnp.mean(dlt * dlt, axis=0, keepdims=True)
    yn = dlt * lax.rsqrt(var + 1e-5) * g_ref[...] + be_ref[...]
    out_ref[...] = jnp.maximum(yn, 0.0) + h


_tc_layer_call = pl.pallas_call(
    _tc_layer_body,
    out_shape=jax.ShapeDtypeStruct((N, D), jnp.float32),
)


def _tc_final_body(agg_ref, cnt_ref, h_ref, wl_ref, bl_ref, wr_ref,
                   w1a_ref, w1b_ref, b1_ref,
                   z_ref, p_ref, q_ref):
    mean = _mean_from_parts(agg_ref, cnt_ref)
    z = (jnp.dot(mean, wl_ref[...], preferred_element_type=jnp.float32)
         + bl_ref[...]
         + jnp.dot(h_ref[...], wr_ref[...], preferred_element_type=jnp.float32))
    z_ref[...] = z
    p_ref[...] = jnp.dot(z, w1a_ref[...], preferred_element_type=jnp.float32)
    q_ref[...] = (jnp.dot(z, w1b_ref[...], preferred_element_type=jnp.float32)
                  + b1_ref[...])


_tc_final_call = pl.pallas_call(
    _tc_final_body,
    out_shape=(jax.ShapeDtypeStruct((N, D), jnp.float32),
               jax.ShapeDtypeStruct((N, D), jnp.float32),
               jax.ShapeDtypeStruct((N, D), jnp.float32)),
)


def kernel(x, edge_index, pos_edge_index, neg_edge_index,
           Wl0, bl0, Wr0, Wl1, bl1, Wr1, Wl2, bl2, Wr2,
           g0, be0, g1, be1, ep_W1, ep_b1, ep_W2, ep_b2):
    i32 = jnp.int32
    src = edge_index[0].astype(i32)
    dst = edge_index[1].astype(i32)
    src_p = jnp.concatenate([src, jnp.zeros((EPAD - E,), i32)])
    dst_p = jnp.concatenate([dst, jnp.full((EPAD - E,), DUMMY, i32)])
    zeros = jnp.zeros((NPAD, D), jnp.float32)
    ones_mat = jnp.ones((N, D), jnp.float32)

    bl0r, bl1r, bl2r = bl0.reshape(1, D), bl1.reshape(1, D), bl2.reshape(1, D)
    g0r, be0r = g0.reshape(1, D), be0.reshape(1, D)
    g1r, be1r = g1.reshape(1, D), be1.reshape(1, D)

    cntp = _agg_call(src_p, dst_p, ones_mat, zeros)
    agg0 = _agg_call(src_p, dst_p, x, zeros)
    h0 = _tc_layer_call(agg0, cntp, x, Wl0, bl0r, Wr0, g0r, be0r)
    agg1 = _agg_call(src_p, dst_p, h0, zeros)
    h1 = _tc_layer_call(agg1, cntp, h0, Wl1, bl1r, Wr1, g1r, be1r)
    agg2 = _agg_call(src_p, dst_p, h1, zeros)
    z, p, q = _tc_final_call(agg2, cntp, h1, Wl2, bl2r, Wr2,
                             ep_W1[:D, :], ep_W1[D:, :], ep_b1.reshape(1, D))

    es = jnp.concatenate([pos_edge_index[0], neg_edge_index[0],
                          jnp.zeros((EDPAD - ED,), i32)])
    ed = jnp.concatenate([pos_edge_index[1], neg_edge_index[1],
                          jnp.zeros((EDPAD - ED,), i32)])
    w2 = ep_W2[:, 0]
    b2v = jnp.broadcast_to(ep_b2, (16,)).astype(jnp.float32)
    scores = _decode_call(es, ed, p, q, w2, b2v)
    return z, scores[:100000], scores[100000:200000]
